# baseline (speedup 1.0000x reference)
import jax
import jax.numpy as jnp
from jax.experimental import pallas as pl
from jax.experimental.pallas import tpu as pltpu

NUM_TOKENS = 32768
HIDDEN = 768
NUM_EXPERTS = 64

BLOCK = 1024
NSPLIT = 3
CHUNK = HIDDEN // NSPLIT


def _router_block(*refs):
    x_refs = refs[:NSPLIT]
    w_refs = refs[NSPLIT:2 * NSPLIT]
    scores_ref, w_ref, i_ref = refs[2 * NSPLIT:]
    logits = jnp.dot(x_refs[0][...], w_refs[0][...],
                     preferred_element_type=jnp.float32)
    for j in range(1, NSPLIT):
        logits += jnp.dot(x_refs[j][...], w_refs[j][...],
                          preferred_element_type=jnp.float32)
    m = jnp.max(logits, axis=-1, keepdims=True)
    e = jnp.exp(logits - m)
    s = jnp.sum(e, axis=-1, keepdims=True)
    scores_ref[...] = e / s
    w_ref[...] = 1.0 / s
    lane = jax.lax.broadcasted_iota(jnp.int32, logits.shape, 1).astype(jnp.float32)
    hit = jnp.where(logits == m, lane, float(NUM_EXPERTS))
    i_ref[...] = jnp.min(hit, axis=-1, keepdims=True).astype(jnp.int32)


@jax.jit
def _router(x, Wt):
    n_blocks = NUM_TOKENS // BLOCK
    x_specs = [
        pl.BlockSpec((BLOCK, CHUNK), lambda i, j=j: (i, j))
        for j in range(NSPLIT)
    ]
    w_specs = [
        pl.BlockSpec((CHUNK, NUM_EXPERTS), lambda i, j=j: (j, 0))
        for j in range(NSPLIT)
    ]
    scores, w, idx = pl.pallas_call(
        _router_block,
        grid=(n_blocks,),
        in_specs=x_specs + w_specs,
        out_specs=[
            pl.BlockSpec((BLOCK, NUM_EXPERTS), lambda i: (i, 0)),
            pl.BlockSpec((BLOCK, 1), lambda i: (i, 0)),
            pl.BlockSpec((BLOCK, 1), lambda i: (i, 0)),
        ],
        out_shape=[
            jax.ShapeDtypeStruct((NUM_TOKENS, NUM_EXPERTS), jnp.float32),
            jax.ShapeDtypeStruct((NUM_TOKENS, 1), jnp.float32),
            jax.ShapeDtypeStruct((NUM_TOKENS, 1), jnp.int32),
        ],
        compiler_params=pltpu.CompilerParams(
            dimension_semantics=("parallel",),
        ),
    )(*([x] * NSPLIT + [Wt] * NSPLIT))
    return w, idx, scores


def kernel(x, W):
    x2 = x.reshape(-1, x.shape[-1])
    w, idx, scores = _router(x2, W.T)
    return (w, idx, scores)


# speedup vs baseline: 1.0757x; 1.0757x over previous
import jax
import jax.numpy as jnp
from jax.experimental import pallas as pl
from jax.experimental.pallas import tpu as pltpu

NUM_TOKENS = 32768
HIDDEN = 768
NUM_EXPERTS = 64

BLOCK = 2048
NSPLIT = 3
CHUNK = HIDDEN // NSPLIT


def _router_block(*refs):
    x_refs = refs[:NSPLIT]
    w_refs = refs[NSPLIT:2 * NSPLIT]
    scores_ref, w_ref, i_ref = refs[2 * NSPLIT:]
    logits = jnp.dot(x_refs[0][...], w_refs[0][...],
                     preferred_element_type=jnp.float32)
    for j in range(1, NSPLIT):
        logits += jnp.dot(x_refs[j][...], w_refs[j][...],
                          preferred_element_type=jnp.float32)
    m = jnp.max(logits, axis=-1, keepdims=True)
    e = jnp.exp(logits - m)
    s = jnp.sum(e, axis=-1, keepdims=True)
    scores_ref[...] = e / s
    w_ref[...] = 1.0 / s
    lane = jax.lax.broadcasted_iota(jnp.int32, logits.shape, 1).astype(jnp.float32)
    hit = jnp.where(logits == m, lane, float(NUM_EXPERTS))
    i_ref[...] = jnp.min(hit, axis=-1, keepdims=True).astype(jnp.int32)


@jax.jit
def _router(x, Wt):
    n_blocks = NUM_TOKENS // BLOCK
    x_specs = [
        pl.BlockSpec((BLOCK, CHUNK), lambda i, j=j: (i, j))
        for j in range(NSPLIT)
    ]
    w_specs = [
        pl.BlockSpec((CHUNK, NUM_EXPERTS), lambda i, j=j: (j, 0))
        for j in range(NSPLIT)
    ]
    scores, w, idx = pl.pallas_call(
        _router_block,
        grid=(n_blocks,),
        in_specs=x_specs + w_specs,
        out_specs=[
            pl.BlockSpec((BLOCK, NUM_EXPERTS), lambda i: (i, 0)),
            pl.BlockSpec((BLOCK, 1), lambda i: (i, 0)),
            pl.BlockSpec((BLOCK, 1), lambda i: (i, 0)),
        ],
        out_shape=[
            jax.ShapeDtypeStruct((NUM_TOKENS, NUM_EXPERTS), jnp.float32),
            jax.ShapeDtypeStruct((NUM_TOKENS, 1), jnp.float32),
            jax.ShapeDtypeStruct((NUM_TOKENS, 1), jnp.int32),
        ],
        compiler_params=pltpu.CompilerParams(
            dimension_semantics=("parallel",),
        ),
    )(*([x] * NSPLIT + [Wt] * NSPLIT))
    return w, idx, scores


def kernel(x, W):
    x2 = x.reshape(-1, x.shape[-1])
    w, idx, scores = _router(x2, W.T)
    return (w, idx, scores)


# speedup vs baseline: 1.1161x; 1.0376x over previous
import jax
import jax.numpy as jnp
from jax.experimental import pallas as pl
from jax.experimental.pallas import tpu as pltpu

NUM_TOKENS = 32768
HIDDEN = 768
NUM_EXPERTS = 64

BLOCK = 4096
NSPLIT = 3
CHUNK = HIDDEN // NSPLIT


def _router_block(*refs):
    x_refs = refs[:NSPLIT]
    w_refs = refs[NSPLIT:2 * NSPLIT]
    scores_ref, w_ref, i_ref = refs[2 * NSPLIT:]
    logits = jnp.dot(x_refs[0][...], w_refs[0][...],
                     preferred_element_type=jnp.float32)
    for j in range(1, NSPLIT):
        logits += jnp.dot(x_refs[j][...], w_refs[j][...],
                          preferred_element_type=jnp.float32)
    m = jnp.max(logits, axis=-1, keepdims=True)
    e = jnp.exp(logits - m)
    s = jnp.sum(e, axis=-1, keepdims=True)
    scores_ref[...] = e / s
    w_ref[...] = 1.0 / s
    lane = jax.lax.broadcasted_iota(jnp.int32, logits.shape, 1).astype(jnp.float32)
    hit = jnp.where(logits == m, lane, float(NUM_EXPERTS))
    i_ref[...] = jnp.min(hit, axis=-1, keepdims=True).astype(jnp.int32)


@jax.jit
def _router(x, Wt):
    n_blocks = NUM_TOKENS // BLOCK
    x_specs = [
        pl.BlockSpec((BLOCK, CHUNK), lambda i, j=j: (i, j))
        for j in range(NSPLIT)
    ]
    w_specs = [
        pl.BlockSpec((CHUNK, NUM_EXPERTS), lambda i, j=j: (j, 0))
        for j in range(NSPLIT)
    ]
    scores, w, idx = pl.pallas_call(
        _router_block,
        grid=(n_blocks,),
        in_specs=x_specs + w_specs,
        out_specs=[
            pl.BlockSpec((BLOCK, NUM_EXPERTS), lambda i: (i, 0)),
            pl.BlockSpec((BLOCK, 1), lambda i: (i, 0)),
            pl.BlockSpec((BLOCK, 1), lambda i: (i, 0)),
        ],
        out_shape=[
            jax.ShapeDtypeStruct((NUM_TOKENS, NUM_EXPERTS), jnp.float32),
            jax.ShapeDtypeStruct((NUM_TOKENS, 1), jnp.float32),
            jax.ShapeDtypeStruct((NUM_TOKENS, 1), jnp.int32),
        ],
        compiler_params=pltpu.CompilerParams(
            dimension_semantics=("parallel",),
        ),
    )(*([x] * NSPLIT + [Wt] * NSPLIT))
    return w, idx, scores


def kernel(x, W):
    x2 = x.reshape(-1, x.shape[-1])
    w, idx, scores = _router(x2, W.T)
    return (w, idx, scores)


# speedup vs baseline: 1.1371x; 1.0187x over previous
import jax
import jax.numpy as jnp
from jax.experimental import pallas as pl
from jax.experimental.pallas import tpu as pltpu

NUM_TOKENS = 32768
HIDDEN = 768
NUM_EXPERTS = 64

BLOCK = 1024
DEPTH = 8


def _router_block(x_hbm, wt_ref, scores_ref, w_ref, i_ref, xbuf, sems):
    step = pl.program_id(0)
    nsteps = pl.num_programs(0)

    def copy(block, slot):
        return pltpu.make_async_copy(
            x_hbm.at[pl.ds(block * BLOCK, BLOCK), :],
            xbuf.at[slot],
            sems.at[slot],
        )

    @pl.when(step == 0)
    def _():
        for d in range(DEPTH):
            copy(d, d).start()

    slot = jax.lax.rem(step, DEPTH)
    copy(step, slot).wait()

    logits = jnp.dot(xbuf[slot], wt_ref[...], preferred_element_type=jnp.float32)
    m = jnp.max(logits, axis=-1, keepdims=True)
    e = jnp.exp(logits - m)
    s = jnp.sum(e, axis=-1, keepdims=True)
    scores_ref[...] = e / s
    w_ref[...] = 1.0 / s
    lane = jax.lax.broadcasted_iota(jnp.int32, logits.shape, 1).astype(jnp.float32)
    hit = jnp.where(logits == m, lane, float(NUM_EXPERTS))
    i_ref[...] = jnp.min(hit, axis=-1, keepdims=True).astype(jnp.int32)

    @pl.when(step + DEPTH < nsteps)
    def _():
        copy(step + DEPTH, slot).start()


@jax.jit
def _router(x, Wt):
    n_blocks = NUM_TOKENS // BLOCK
    scores, w, idx = pl.pallas_call(
        _router_block,
        grid=(n_blocks,),
        in_specs=[
            pl.BlockSpec(memory_space=pl.MemorySpace.ANY),
            pl.BlockSpec((HIDDEN, NUM_EXPERTS), lambda i: (0, 0)),
        ],
        out_specs=[
            pl.BlockSpec((BLOCK, NUM_EXPERTS), lambda i: (i, 0)),
            pl.BlockSpec((BLOCK, 1), lambda i: (i, 0)),
            pl.BlockSpec((BLOCK, 1), lambda i: (i, 0)),
        ],
        out_shape=[
            jax.ShapeDtypeStruct((NUM_TOKENS, NUM_EXPERTS), jnp.float32),
            jax.ShapeDtypeStruct((NUM_TOKENS, 1), jnp.float32),
            jax.ShapeDtypeStruct((NUM_TOKENS, 1), jnp.int32),
        ],
        scratch_shapes=[
            pltpu.VMEM((DEPTH, BLOCK, HIDDEN), jnp.float32),
            pltpu.SemaphoreType.DMA((DEPTH,)),
        ],
        compiler_params=pltpu.CompilerParams(
            dimension_semantics=("arbitrary",),
        ),
    )(x, Wt)
    return w, idx, scores


def kernel(x, W):
    x2 = x.reshape(-1, x.shape[-1])
    w, idx, scores = _router(x2, W.T)
    return (w, idx, scores)
